# baseline (speedup 1.0000x reference)
import functools

import jax
import jax.numpy as jnp
from jax import lax
from jax.experimental import pallas as pl
from jax.experimental.pallas import tpu as pltpu
from jax.experimental.pallas import tpu_sc as plsc

B = 16384
D = 128
NC = 2
NS = 16
NW = NC * NS
B_PER_W = B // NW
R = 128
N_CHUNKS = B_PER_W // R

_mesh = plsc.VectorSubcoreMesh(core_axis_name="c", subcore_axis_name="s")


@functools.partial(
    pl.kernel,
    mesh=_mesh,
    out_type=jax.ShapeDtypeStruct((B, D), jnp.float32),
    scratch_types=[
        pltpu.VMEM((B_PER_W,), jnp.int32),
        pltpu.VMEM((R, D), jnp.float32),
        pltpu.VMEM((R, D), jnp.float32),
        pltpu.VMEM((R, D), jnp.float32),
        pltpu.VMEM((R, D), jnp.float32),
        pltpu.SemaphoreType.DMA,
        pltpu.SemaphoreType.DMA,
        pltpu.SemaphoreType.DMA,
        pltpu.SemaphoreType.DMA,
        pltpu.SemaphoreType.DMA,
        pltpu.SemaphoreType.DMA,
    ],
)
def _sc_diag(x_hbm, idx_hbm, emb_hbm, out_hbm, idx_all,
             x_v0, x_v1, r_v0, r_v1, si0, si1, sg0, sg1, so0, so1):
    wid = lax.axis_index("s") * NC + lax.axis_index("c")
    base = wid * B_PER_W
    pltpu.sync_copy(idx_hbm.at[pl.ds(base, B_PER_W)], idx_all)
    xv = [x_v0, x_v1]
    rv = [r_v0, r_v1]
    sin = [si0, si1]
    sg = [sg0, sg1]
    sout = [so0, so1]
    descs = [None, None]
    for ch in range(N_CHUNKS):
        b = ch % 2
        row0 = base + ch * R
        if descs[b] is not None:
            descs[b].wait()
        d_in = pltpu.async_copy(x_hbm.at[pl.ds(row0, R)], xv[b], sin[b])
        d_g = pltpu.async_copy(emb_hbm.at[idx_all.at[pl.ds(ch * R, R)]],
                               rv[b], sg[b])
        d_in.wait()
        d_g.wait()
        descs[b] = pltpu.async_copy(xv[b], out_hbm.at[pl.ds(row0, R)], sout[b])
    for d in descs:
        d.wait()


def kernel(x, condition_idx, embeddings):
    idx = condition_idx.astype(jnp.int32)
    return _sc_diag(x, idx, embeddings)


# speedup vs baseline: 1.5291x; 1.5291x over previous
import functools

import jax
import jax.numpy as jnp
from jax import lax
from jax.experimental import pallas as pl
from jax.experimental.pallas import tpu as pltpu
from jax.experimental.pallas import tpu_sc as plsc

B = 16384
D = 128
NV = 100
NC = 2
NS = 16
NW = NC * NS
B_PER_W = B // NW
R = 128
N_CHUNKS = B_PER_W // R
N_BUF = 2

_mesh = plsc.VectorSubcoreMesh(core_axis_name="c", subcore_axis_name="s")

_scratch = (
    [pltpu.VMEM((B_PER_W,), jnp.int32),
     pltpu.VMEM_SHARED((NV, D), jnp.float32)]
    + [pltpu.VMEM((R, D), jnp.float32) for _ in range(N_BUF)]
    + [pltpu.VMEM((R, D), jnp.float32) for _ in range(N_BUF)]
    + [pltpu.SemaphoreType.DMA for _ in range(N_BUF)]
    + [pltpu.SemaphoreType.DMA for _ in range(N_BUF)]
    + [pltpu.SemaphoreType.DMA for _ in range(N_BUF)]
)


@functools.partial(
    pl.kernel,
    mesh=_mesh,
    out_type=jax.ShapeDtypeStruct((B, D), jnp.float32),
    scratch_types=_scratch,
)
def _sc_embed_add(x_hbm, idx_hbm, emb_hbm, out_hbm, idx_all, emb_v, *bufs):
    x_v = bufs[:N_BUF]
    rows_v = bufs[N_BUF:2 * N_BUF]
    semx = bufs[2 * N_BUF:3 * N_BUF]
    semg = bufs[3 * N_BUF:4 * N_BUF]
    semo = bufs[4 * N_BUF:5 * N_BUF]

    wid = lax.axis_index("s") * NC + lax.axis_index("c")
    base = wid * B_PER_W
    sid = lax.axis_index("s")

    @pl.when(sid == 0)
    def _():
        pltpu.sync_copy(emb_hbm, emb_v)

    pltpu.sync_copy(idx_hbm.at[pl.ds(base, B_PER_W)], idx_all)
    plsc.subcore_barrier()

    out_descs = [None for _ in range(N_BUF)]
    for ch in range(N_CHUNKS):
        b = ch % N_BUF
        row0 = base + ch * R
        if out_descs[b] is not None:
            out_descs[b].wait()
        d_x = pltpu.async_copy(x_hbm.at[pl.ds(row0, R)], x_v[b], semx[b])
        d_g = pltpu.async_copy(emb_v.at[idx_all.at[pl.ds(ch * R, R)]],
                               rows_v[b], semg[b])
        d_x.wait()
        d_g.wait()

        xbuf = x_v[b]
        rbuf = rows_v[b]

        def add_row(r, carry):
            for j in range(D // 16):
                sl = pl.ds(j * 16, 16)
                plsc.addupdate(xbuf.at[r, sl], rbuf[r, sl])
            return carry

        lax.fori_loop(0, R, add_row, 0)
        out_descs[b] = pltpu.async_copy(xbuf, out_hbm.at[pl.ds(row0, R)],
                                        semo[b])
    for d in out_descs:
        d.wait()


def kernel(x, condition_idx, embeddings):
    idx = condition_idx.astype(jnp.int32)
    return _sc_embed_add(x, idx, embeddings)


# speedup vs baseline: 1.7071x; 1.1164x over previous
import functools

import jax
import jax.numpy as jnp
from jax import lax
from jax.experimental import pallas as pl
from jax.experimental.pallas import tpu as pltpu
from jax.experimental.pallas import tpu_sc as plsc

B = 16384
D = 128
NV = 100
NC = 2
NS = 16
NW = NC * NS
B_PER_W = B // NW
R = 128
N_CHUNKS = B_PER_W // R
N_XBUF = 3

_mesh = plsc.VectorSubcoreMesh(core_axis_name="c", subcore_axis_name="s")

_scratch = (
    [pltpu.VMEM((B_PER_W,), jnp.int32),
     pltpu.VMEM_SHARED((NV, D), jnp.float32)]
    + [pltpu.VMEM((R, D), jnp.float32) for _ in range(N_XBUF)]
    + [pltpu.VMEM((R, D), jnp.float32) for _ in range(N_CHUNKS)]
    + [pltpu.SemaphoreType.DMA for _ in range(N_XBUF)]
    + [pltpu.SemaphoreType.DMA for _ in range(N_CHUNKS)]
    + [pltpu.SemaphoreType.DMA for _ in range(N_XBUF)]
)


@functools.partial(
    pl.kernel,
    mesh=_mesh,
    out_type=jax.ShapeDtypeStruct((B, D), jnp.float32),
    scratch_types=_scratch,
)
def _sc_embed_add(x_hbm, idx_hbm, emb_hbm, out_hbm, idx_all, emb_sh, *bufs):
    x_v = bufs[:N_XBUF]
    rows_v = bufs[N_XBUF:N_XBUF + N_CHUNKS]
    semx = bufs[N_XBUF + N_CHUNKS:2 * N_XBUF + N_CHUNKS]
    semg = bufs[2 * N_XBUF + N_CHUNKS:2 * N_XBUF + 2 * N_CHUNKS]
    semo = bufs[2 * N_XBUF + 2 * N_CHUNKS:]

    wid = lax.axis_index("s") * NC + lax.axis_index("c")
    base = wid * B_PER_W
    sid = lax.axis_index("s")

    @pl.when(sid == 0)
    def _():
        pltpu.sync_copy(emb_hbm, emb_sh)

    pltpu.sync_copy(idx_hbm.at[pl.ds(base, B_PER_W)], idx_all)
    plsc.subcore_barrier()

    g_descs = [
        pltpu.async_copy(emb_sh.at[idx_all.at[pl.ds(ch * R, R)]],
                         rows_v[ch], semg[ch])
        for ch in range(N_CHUNKS)
    ]
    x_descs = [None for _ in range(N_CHUNKS)]
    out_descs = [None for _ in range(N_XBUF)]

    def issue_x(ch):
        b = ch % N_XBUF
        if out_descs[b] is not None:
            out_descs[b].wait()
        x_descs[ch] = pltpu.async_copy(x_hbm.at[pl.ds(base + ch * R, R)],
                                       x_v[b], semx[b])

    issue_x(0)
    for ch in range(N_CHUNKS):
        b = ch % N_XBUF
        if ch + 1 < N_CHUNKS:
            issue_x(ch + 1)
        x_descs[ch].wait()
        g_descs[ch].wait()

        xbuf = x_v[b]
        rbuf = rows_v[ch]

        def add_row(r, carry):
            for j in range(D // 16):
                sl = pl.ds(j * 16, 16)
                plsc.addupdate(xbuf.at[r, sl], rbuf[r, sl])
            return carry

        lax.fori_loop(0, R, add_row, 0)
        out_descs[b] = pltpu.async_copy(xbuf,
                                        out_hbm.at[pl.ds(base + ch * R, R)],
                                        semo[b])
    for d in out_descs:
        if d is not None:
            d.wait()


def kernel(x, condition_idx, embeddings):
    idx = condition_idx.astype(jnp.int32)
    return _sc_embed_add(x, idx, embeddings)


# speedup vs baseline: 1.7714x; 1.0377x over previous
import functools

import jax
import jax.numpy as jnp
from jax import lax
from jax.experimental import pallas as pl
from jax.experimental.pallas import tpu as pltpu
from jax.experimental.pallas import tpu_sc as plsc

B = 16384
D = 128
NV = 100
NC = 2
NS = 16
NW = NC * NS
B_PER_W = B // NW
R = 128
N_CHUNKS = B_PER_W // R
N_XBUF = 3

_mesh = plsc.VectorSubcoreMesh(core_axis_name="c", subcore_axis_name="s")

_scratch = (
    [pltpu.VMEM((B_PER_W,), jnp.int32),
     pltpu.VMEM_SHARED((NV, D), jnp.float32)]
    + [pltpu.VMEM((R, D), jnp.float32) for _ in range(N_XBUF)]
    + [pltpu.SemaphoreType.DMA for _ in range(N_XBUF)]
    + [pltpu.SemaphoreType.DMA for _ in range(N_XBUF)]
    + [pltpu.SemaphoreType.DMA for _ in range(N_XBUF)]
)


@functools.partial(
    pl.kernel,
    mesh=_mesh,
    out_type=jax.ShapeDtypeStruct((B, D), jnp.float32),
    scratch_types=_scratch,
)
def _sc_embed_add(x_hbm, idx_hbm, emb_hbm, out_hbm, idx_all, emb_sh, *bufs):
    x_v = bufs[:N_XBUF]
    semx = bufs[N_XBUF:2 * N_XBUF]
    semg = bufs[2 * N_XBUF:3 * N_XBUF]
    semo = bufs[3 * N_XBUF:]

    wid = lax.axis_index("s") * NC + lax.axis_index("c")
    base = wid * B_PER_W
    sid = lax.axis_index("s")

    @pl.when(sid == 0)
    def _():
        pltpu.sync_copy(emb_hbm, emb_sh)

    pltpu.sync_copy(idx_hbm.at[pl.ds(base, B_PER_W)], idx_all)
    plsc.subcore_barrier()

    x_descs = [None for _ in range(N_CHUNKS)]
    out_descs = [None for _ in range(N_XBUF)]

    def issue_x(ch):
        b = ch % N_XBUF
        if out_descs[b] is not None:
            out_descs[b].wait()
        x_descs[ch] = pltpu.async_copy(x_hbm.at[pl.ds(base + ch * R, R)],
                                       x_v[b], semx[b])

    issue_x(0)
    for ch in range(N_CHUNKS):
        b = ch % N_XBUF
        if ch + 1 < N_CHUNKS:
            issue_x(ch + 1)
        x_descs[ch].wait()
        g = pltpu.async_copy(emb_sh.at[idx_all.at[pl.ds(ch * R, R)]],
                             x_v[b], semg[b], add=True)
        g.wait()
        out_descs[b] = pltpu.async_copy(x_v[b],
                                        out_hbm.at[pl.ds(base + ch * R, R)],
                                        semo[b])
    for d in out_descs:
        if d is not None:
            d.wait()


def kernel(x, condition_idx, embeddings):
    idx = condition_idx.astype(jnp.int32)
    return _sc_embed_add(x, idx, embeddings)
